# baseline (speedup 1.0000x reference)
import functools

import jax
import jax.numpy as jnp
from jax import lax
from jax.experimental import pallas as pl
from jax.experimental.pallas import tpu as pltpu
from jax.experimental.pallas import tpu_sc as plsc

BATCH = 16384
D_IN = 128
FEAT_DIM = 64

_NC = 2
_NS = 16
_NW = _NC * _NS
_BPW = BATCH // _NW


def _matmul_body(feat_ref, w_ref, b_ref, out_ref):
    out_ref[...] = (
        lax.dot_general(
            feat_ref[...], w_ref[...],
            (((1,), (1,)), ((), ())),
            preferred_element_type=jnp.float32,
        )
        + b_ref[...]
    )


def _tc_projection(feat, W, b):
    blk = 2048
    grid = BATCH // blk
    return pl.pallas_call(
        _matmul_body,
        grid=(grid,),
        in_specs=[
            pl.BlockSpec((blk, D_IN), lambda i: (i, 0)),
            pl.BlockSpec((FEAT_DIM, D_IN), lambda i: (0, 0)),
            pl.BlockSpec((1, FEAT_DIM), lambda i: (0, 0)),
        ],
        out_specs=pl.BlockSpec((blk, FEAT_DIM), lambda i: (i, 0)),
        out_shape=jax.ShapeDtypeStruct((BATCH, FEAT_DIM), jnp.float32),
    )(feat, W, b.reshape(1, FEAT_DIM))


def _sc_assemble_body(proj_hbm, idx_hbm, table_hbm, out_hbm,
                      idx_v, rows_v, proj_v, sem):
    wid = lax.axis_index("s") * _NC + lax.axis_index("c")
    base = wid * _BPW
    pltpu.sync_copy(idx_hbm.at[pl.ds(base, _BPW)], idx_v)
    gather = pltpu.async_copy(table_hbm.at[idx_v], rows_v, sem)
    pltpu.sync_copy(proj_hbm.at[pl.ds(base, _BPW)], proj_v)
    pltpu.sync_copy(proj_v, out_hbm.at[pl.ds(base, _BPW), pl.ds(0, FEAT_DIM)])
    gather.wait()
    pltpu.sync_copy(rows_v, out_hbm.at[pl.ds(base, _BPW),
                                       pl.ds(FEAT_DIM, FEAT_DIM)])


@functools.partial(
    pl.kernel,
    out_type=jax.ShapeDtypeStruct((BATCH, D_IN), jnp.float32),
    mesh=plsc.VectorSubcoreMesh(core_axis_name="c", subcore_axis_name="s"),
    compiler_params=pltpu.CompilerParams(use_tc_tiling_on_sc=False),
    scratch_types=[
        pltpu.VMEM((_BPW,), jnp.int32),
        pltpu.VMEM((_BPW, FEAT_DIM), jnp.float32),
        pltpu.VMEM((_BPW, FEAT_DIM), jnp.float32),
        pltpu.SemaphoreType.DMA,
    ],
)
def _sc_assemble(proj_hbm, idx_hbm, table_hbm, out_hbm,
                 idx_v, rows_v, proj_v, sem):
    _sc_assemble_body(proj_hbm, idx_hbm, table_hbm, out_hbm,
                      idx_v, rows_v, proj_v, sem)


def kernel(feat, idx, W, b, table):
    proj = _tc_projection(feat, W, b)
    return _sc_assemble(proj, idx.astype(jnp.int32), table)


# speedup vs baseline: 1.5476x; 1.5476x over previous
import functools

import jax
import jax.numpy as jnp
from jax import lax
from jax.experimental import pallas as pl
from jax.experimental.pallas import tpu as pltpu
from jax.experimental.pallas import tpu_sc as plsc

BATCH = 16384
D_IN = 128
FEAT_DIM = 64

_NC = 2
_NS = 16
_NW = _NC * _NS
_BPW = BATCH // _NW
_CH = 16


def _extract_rows(buf, idx_sm, rows_v, g0, cols):
    for i in range(_CH):
        g = g0 + i
        r = idx_sm[g]
        row = i * 8 + (r & 7)
        for k in range(4):
            rows_v[g, pl.ds(16 * k, 16)] = buf[row, pl.ds(16 * k, 16)]


def _fire_chunk(table_hbm, idx_sm, buf, sem, g0):
    copies = []
    for i in range(_CH):
        r = idx_sm[g0 + i]
        base8 = pl.multiple_of((r >> 3) << 3, 8)
        copies.append(
            pltpu.async_copy(table_hbm.at[pl.ds(base8, 8)],
                             buf.at[pl.ds(i * 8, 8)], sem))
    return copies


def _sc_gather_body(idx_hbm, table_hbm, emb_hbm,
                    idx_v, idx_sm, buf_a, buf_b, rows_v, sem_a, sem_b):
    wid = lax.axis_index("s") * _NC + lax.axis_index("c")
    base = wid * _BPW
    pltpu.sync_copy(idx_hbm.at[pl.ds(base, _BPW)], idx_v)
    lanes = lax.iota(jnp.int32, 16)

    def smemify(t, carry):
        v = idx_v[pl.ds(t * 16, 16)]
        for j in range(16):
            idx_sm[t * 16 + j] = v[j]
        return carry

    lax.fori_loop(0, _BPW // 16, smemify, 0)
    cols = [lax.iota(jnp.int32, 16) + 16 * k for k in range(4)]

    def chunk_pair(j, carry):
        g0 = j * (2 * _CH)
        ca = _fire_chunk(table_hbm, idx_sm, buf_a, sem_a, g0)
        cb = _fire_chunk(table_hbm, idx_sm, buf_b, sem_b, g0 + _CH)
        for c in ca:
            c.wait()
        _extract_rows(buf_a, idx_sm, rows_v, g0, cols)
        for c in cb:
            c.wait()
        _extract_rows(buf_b, idx_sm, rows_v, g0 + _CH, cols)
        return carry

    lax.fori_loop(0, _BPW // (2 * _CH), chunk_pair, 0)
    pltpu.sync_copy(rows_v, emb_hbm.at[pl.ds(base, _BPW)])


@functools.partial(
    pl.kernel,
    out_type=jax.ShapeDtypeStruct((BATCH, FEAT_DIM), jnp.float32),
    mesh=plsc.VectorSubcoreMesh(core_axis_name="c", subcore_axis_name="s"),
    scratch_types=[
        pltpu.VMEM((_BPW,), jnp.int32),
        pltpu.SMEM((_BPW,), jnp.int32),
        pltpu.VMEM((_CH * 8, FEAT_DIM), jnp.float32),
        pltpu.VMEM((_CH * 8, FEAT_DIM), jnp.float32),
        pltpu.VMEM((_BPW, FEAT_DIM), jnp.float32),
        pltpu.SemaphoreType.DMA,
        pltpu.SemaphoreType.DMA,
    ],
)
def _sc_gather(idx_hbm, table_hbm, emb_hbm,
               idx_v, idx_sm, buf_a, buf_b, rows_v, sem_a, sem_b):
    _sc_gather_body(idx_hbm, table_hbm, emb_hbm,
                    idx_v, idx_sm, buf_a, buf_b, rows_v, sem_a, sem_b)


def _tc_body(feat_ref, w_ref, b_ref, emb_ref, out_ref):
    proj = (
        lax.dot_general(
            feat_ref[...], w_ref[...],
            (((1,), (1,)), ((), ())),
            preferred_element_type=jnp.float32,
        )
        + b_ref[...]
    )
    out_ref[...] = jnp.concatenate([proj, emb_ref[...]], axis=-1)


def _tc_project_concat(feat, W, b, emb):
    blk = 2048
    grid = BATCH // blk
    return pl.pallas_call(
        _tc_body,
        grid=(grid,),
        in_specs=[
            pl.BlockSpec((blk, D_IN), lambda i: (i, 0)),
            pl.BlockSpec((FEAT_DIM, D_IN), lambda i: (0, 0)),
            pl.BlockSpec((1, FEAT_DIM), lambda i: (0, 0)),
            pl.BlockSpec((blk, FEAT_DIM), lambda i: (i, 0)),
        ],
        out_specs=pl.BlockSpec((blk, D_IN), lambda i: (i, 0)),
        out_shape=jax.ShapeDtypeStruct((BATCH, D_IN), jnp.float32),
    )(feat, W, b.reshape(1, FEAT_DIM), emb)


def kernel(feat, idx, W, b, table):
    emb = _sc_gather(idx.astype(jnp.int32), table)
    return _tc_project_concat(feat, W, b, emb)
